# baseline (speedup 1.0000x reference)
import functools

import jax
import jax.numpy as jnp
from jax import lax
from jax.experimental import pallas as pl
from jax.experimental.pallas import tpu as pltpu
from jax.experimental.pallas import tpu_sc as plsc

_V = 1000000
_B = 16384
_D = 32
_W = _D // 2
_L = 20
_NNEG = 50
_R = 128
_NW = 32
_PER_W = _B // _NW
_E = 16
_CHUNKS = _PER_W // _E

_RV = 121

_PK_COLS = 640
_PK_CHUNKS = 50

_sc_mesh = plsc.VectorSubcoreMesh(core_axis_name="c", subcore_axis_name="s")


def _sc_pack_body(tbl_hbm, out_hbm, in_a, in_b, out_a, out_b,
                  s_in_a, s_in_b, s_out_a, s_out_b):
    in_v = (in_a, in_b)
    out_v = (out_a, out_b)
    s_in = (s_in_a, s_in_b)
    s_out = (s_out_a, s_out_b)
    wid = lax.axis_index("s") * 2 + lax.axis_index("c")
    lane = lax.iota(jnp.int32, 16)

    def base_of(c):
        k = c * _NW + wid
        return jnp.minimum(k * _PK_COLS, _V - _PK_COLS)

    def issue_in(c, b):
        pltpu.async_copy(
            tbl_hbm.at[:, pl.ds(base_of(c), _PK_COLS)], in_v[b], s_in[b])

    def wait_in(b):
        pltpu.make_async_copy(
            tbl_hbm.at[:, pl.ds(0, _PK_COLS)], in_v[b], s_in[b]).wait()

    def wait_out(b):
        pltpu.make_async_copy(
            out_v[b], out_hbm.at[pl.ds(0, _PK_COLS)], s_out[b]).wait()

    issue_in(0, 0)

    def body(g, _):
        for b in range(2):
            c = 2 * g + b
            issue_in(c + 1, 1 - b)
            wait_in(b)

            @pl.when(c >= 2)
            def _():
                wait_out(b)

            def vg_body(i, _):
                v0 = i * 16
                vvec = jnp.full((16,), v0, jnp.int32) + lane
                for d in range(_W):
                    a = in_v[b][d, pl.ds(v0, 16)]
                    y = in_v[b][d + _W, pl.ds(v0, 16)]
                    w = plsc.bitcast(
                        plsc.pack(a, y, format=plsc.PackFormat.INTERLEAVED),
                        jnp.int32)
                    plsc.store_scatter(
                        out_v[b], [vvec, jnp.full((16,), d, jnp.int32)], w)
                return 0

            lax.fori_loop(0, _PK_COLS // 16, vg_body, 0)
            pltpu.async_copy(
                out_v[b], out_hbm.at[pl.ds(base_of(c), _PK_COLS)], s_out[b])
        return 0

    lax.fori_loop(0, _PK_CHUNKS // 2, body, 0)
    wait_in(0)
    wait_out(0)
    wait_out(1)


def _sc_pack(tbl_t):
    kfn = functools.partial(
        pl.kernel,
        mesh=_sc_mesh,
        out_type=jax.ShapeDtypeStruct((_V, _W), jnp.int32),
        scratch_types=(
            [pltpu.VMEM((_D, _PK_COLS), jnp.float32)] * 2
            + [pltpu.VMEM((_PK_COLS, _W), jnp.int32)] * 2
            + [pltpu.SemaphoreType.DMA] * 4
        ),
        compiler_params=pltpu.CompilerParams(
            needs_layout_passes=False, use_tc_tiling_on_sc=False),
    )(_sc_pack_body)
    return kfn(tbl_t)


def _sc_scores(cand_hbm, ctx_hbm, u_pos_hbm, ctx_idx_hbm, out_hbm,
               u_idx_a, u_idx_b, idx_a, idx_b, u_rows_a, u_rows_b,
               rows_a, rows_b, scores_a, scores_b,
               s_idx_a, s_idx_b, s_u_a, s_u_b, s_r_a, s_r_b):
    u_idx = (u_idx_a, u_idx_b)
    idx_v = (idx_a, idx_b)
    u_rows = (u_rows_a, u_rows_b)
    rows_v = (rows_a, rows_b)
    scores_v = (scores_a, scores_b)
    s_idx = (s_idx_a, s_idx_b)
    s_u = (s_u_a, s_u_b)
    s_r = (s_r_a, s_r_b)

    wid = lax.axis_index("s") * 2 + lax.axis_index("c")
    lane = lax.iota(jnp.int32, 16)

    def base_of(c):
        return wid * _PER_W + jnp.minimum(c, _CHUNKS - 1) * _E

    def issue_idx(c, b):
        base = base_of(c)
        pltpu.async_copy(u_pos_hbm.at[pl.ds(base, _E)], u_idx[b], s_idx[b])
        pltpu.async_copy(ctx_idx_hbm.at[pl.ds(base, _E)], idx_v[b], s_idx[b])

    def wait_idx(b):
        pltpu.make_async_copy(
            u_pos_hbm.at[pl.ds(0, _E)], u_idx[b], s_idx[b]).wait()
        pltpu.make_async_copy(
            ctx_idx_hbm.at[pl.ds(0, _E)], idx_v[b], s_idx[b]).wait()

    def issue_rows(b):
        pltpu.async_copy(cand_hbm.at[u_idx[b]], u_rows[b], s_u[b])
        for e in range(_E):
            pltpu.async_copy(ctx_hbm.at[idx_v[b].at[e]],
                             rows_v[b].at[pl.ds(e * _R, _RV)], s_r[b])

    def wait_rows(b):
        pltpu.make_async_copy(
            cand_hbm.at[u_idx[b]], u_rows[b], s_u[b]).wait()
        for e in range(_E):
            pltpu.make_async_copy(
                ctx_hbm.at[idx_v[b].at[e]],
                rows_v[b].at[pl.ds(e * _R, _RV)], s_r[b]).wait()

    def compute(c, b):
        for e in range(_E):
            rowids = [jnp.full((16,), e * _R + g * 16, jnp.int32) + lane
                      for g in range(8)]
            e_splat = jnp.full((16,), e, jnp.int32)

            def d_body(d, accs):
                d_splat = jnp.full((16,), d, jnp.int32)
                uw = plsc.load_gather(u_rows[b], [e_splat, d_splat])
                ub = plsc.bitcast(uw, jnp.bfloat16)
                new = []
                for g in range(8):
                    w = plsc.load_gather(rows_v[b], [rowids[g], d_splat])
                    vb = plsc.bitcast(w, jnp.bfloat16)
                    p = vb * ub
                    lo, hi = plsc.unpack(p, format=plsc.PackFormat.INTERLEAVED)
                    new.append(accs[g] + (lo + hi))
                return tuple(new)

            accs = lax.fori_loop(
                0, _W, d_body,
                tuple(jnp.zeros((16,), jnp.float32) for _ in range(8)))
            for g in range(8):
                scores_v[b][e, pl.ds(g * 16, 16)] = accs[g]

        pltpu.sync_copy(scores_v[b], out_hbm.at[pl.ds(base_of(c), _E)])

    issue_idx(0, 0)
    wait_idx(0)
    issue_rows(0)
    issue_idx(1, 1)

    def body(g, _):
        for b in range(2):
            c = 2 * g + b
            wait_idx(1 - b)
            issue_rows(1 - b)
            wait_rows(b)
            issue_idx(c + 2, b)
            compute(c, b)
        return 0

    lax.fori_loop(0, _CHUNKS // 2, body, 0)
    wait_rows(0)
    wait_idx(1)


def _sc_call(cand_embed, ctx_pk, u_pos, ctx_idx):
    kfn = functools.partial(
        pl.kernel,
        mesh=_sc_mesh,
        out_type=jax.ShapeDtypeStruct((_B, _R), jnp.float32),
        scratch_types=(
            [pltpu.VMEM((_E,), jnp.int32)] * 2
            + [pltpu.VMEM((_E, _RV), jnp.int32)] * 2
            + [pltpu.VMEM((_E, _W), jnp.int32)] * 2
            + [pltpu.VMEM((_E * _R, _W), jnp.int32)] * 2
            + [pltpu.VMEM((_E, _R), jnp.float32)] * 2
            + [pltpu.SemaphoreType.DMA] * 6
        ),
        compiler_params=pltpu.CompilerParams(
            needs_layout_passes=False, use_tc_tiling_on_sc=False),
    )(_sc_scores)
    return kfn(cand_embed, ctx_pk, u_pos, ctx_idx)


def _tc_loss_body(scores_ref, vpos_ref, out_ref):
    s = scores_ref[...]
    vp = vpos_ref[...]
    mask = (vp != 0).astype(jnp.float32)

    def logsig(x):
        return jnp.minimum(x, 0.0) - jnp.log1p(jnp.exp(-jnp.abs(x)))

    s_pos = jnp.sum(s[:, :_L] * mask, axis=1)
    s_book = s[:, _L]
    neg = s[:, _L + 1:_L + 1 + 2 * _NNEG]
    loss = -(logsig(s_pos) + logsig(s_book)
             + jnp.sum(logsig(-neg), axis=1))
    out_ref[...] = loss


def _tc_loss(scores, v_pos):
    bs = 2048
    return pl.pallas_call(
        _tc_loss_body,
        grid=(_B // bs,),
        in_specs=[
            pl.BlockSpec((bs, _R), lambda i: (i, 0)),
            pl.BlockSpec((bs, _L), lambda i: (i, 0)),
        ],
        out_specs=pl.BlockSpec((bs,), lambda i: (i,)),
        out_shape=jax.ShapeDtypeStruct((_B,), jnp.float32),
    )(scores, v_pos)


def kernel(u_pos, v_pos, book_pos, v_neg_city, v_neg_country,
           cand_embed, contx_embed):
    ctx_idx = jnp.concatenate(
        [v_pos, book_pos[:, None], v_neg_city, v_neg_country], axis=1)
    cand_pk = _sc_pack(cand_embed.T)
    ctx_pk = _sc_pack(contx_embed.T)
    scores = _sc_call(cand_pk, ctx_pk, u_pos, ctx_idx)
    return _tc_loss(scores, v_pos)


# speedup vs baseline: 7.5163x; 7.5163x over previous
import functools

import jax
import jax.numpy as jnp
from jax import lax
from jax.experimental import pallas as pl
from jax.experimental.pallas import tpu as pltpu
from jax.experimental.pallas import tpu_sc as plsc

_V = 1000000
_B = 16384
_D = 32
_W = _D // 2
_L = 20
_NNEG = 50
_R = 128
_NW = 32
_PER_W = _B // _NW
_E = 16
_CHUNKS = _PER_W // _E

_RV = 121

_PK_COLS = 640
_PK_CHUNKS = 50
_V_MAIN = 999680
_TAILA = 256
_VB = _V_MAIN + _TAILA

_sc_mesh = plsc.VectorSubcoreMesh(core_axis_name="c", subcore_axis_name="s")


def _sc_pack_body(tbl_hbm, tail_hbm, out_hbm, in_a, in_b, out_a, out_b,
                  s_in_a, s_in_b, s_out_a, s_out_b):
    in_v = (in_a, in_b)
    out_v = (out_a, out_b)
    s_in = (s_in_a, s_in_b)
    s_out = (s_out_a, s_out_b)
    wid = lax.axis_index("s") * 2 + lax.axis_index("c")
    lane = lax.iota(jnp.int32, 16)

    def base_of(c):
        k = c * _NW + wid
        return jnp.minimum(k * _PK_COLS, _V_MAIN - _PK_COLS)

    def issue_in(c, b):
        pltpu.async_copy(
            tbl_hbm.at[:, :, pl.ds(base_of(c), _PK_COLS)], in_v[b], s_in[b])

    def wait_in(b):
        pltpu.make_async_copy(
            tbl_hbm.at[:, :, pl.ds(0, _PK_COLS)], in_v[b], s_in[b]).wait()

    def wait_out(b):
        pltpu.make_async_copy(
            out_v[b], out_hbm.at[pl.ds(0, _PK_COLS * _W)], s_out[b]).wait()

    issue_in(0, 0)

    def body(g, _):
        for b in range(2):
            c = 2 * g + b
            issue_in(c + 1, 1 - b)
            wait_in(b)

            @pl.when(c >= 2)
            def _():
                wait_out(b)

            lax.fori_loop(0, _PK_COLS // 16, _pack_vg(in_v[b], out_v[b], lane), 0)
            pltpu.async_copy(
                out_v[b],
                out_hbm.at[pl.ds(base_of(c) * _W, _PK_COLS * _W)], s_out[b])
        return 0

    lax.fori_loop(0, _PK_CHUNKS // 2, body, 0)
    wait_in(0)
    wait_out(0)
    wait_out(1)

    @pl.when(wid == 0)
    def _():
        pltpu.async_copy(
            tbl_hbm.at[:, :, pl.ds(_V_MAIN, _TAILA)],
            in_v[0].at[:, :, pl.ds(0, _TAILA)], s_in[0]).wait()
        lax.fori_loop(0, _TAILA // 16, _pack_vg(in_v[0], out_v[0], lane), 0)
        pltpu.async_copy(
            out_v[0].at[pl.ds(0, _TAILA * _W)],
            out_hbm.at[pl.ds(_V_MAIN * _W, _TAILA * _W)], s_out[0]).wait()
        pltpu.async_copy(
            tail_hbm, in_v[0].at[:, :, pl.ds(0, 128)], s_in[0]).wait()
        lax.fori_loop(0, (_V - _VB) // 16, _pack_vg(in_v[0], out_v[0], lane), 0)
        pltpu.async_copy(
            out_v[0].at[pl.ds(0, (_V - _VB) * _W)],
            out_hbm.at[pl.ds(_VB * _W, (_V - _VB) * _W)], s_out[0]).wait()


def _pack_vg(in_ref, out_ref, lane):
    def vg_body(i, _):
        v0 = i * 16
        idx0 = (jnp.full((16,), v0, jnp.int32) + lane) * _W
        for d in range(_W):
            a = in_ref[d // 8, d % 8, pl.ds(v0, 16)]
            y = in_ref[(d + _W) // 8, (d + _W) % 8, pl.ds(v0, 16)]
            w = plsc.bitcast(
                plsc.pack(a, y, format=plsc.PackFormat.INTERLEAVED),
                jnp.int32)
            plsc.store_scatter(out_ref, [idx0 + d], w)
        return 0
    return vg_body


def _sc_pack(tbl):
    tbl3 = jnp.reshape(tbl.T, (4, 8, _V))
    tail3 = jnp.reshape(
        jnp.pad(tbl.T[:, _VB:], ((0, 0), (0, 128 - (_V - _VB)))),
        (4, 8, 128))
    kfn = functools.partial(
        pl.kernel,
        mesh=_sc_mesh,
        out_type=jax.ShapeDtypeStruct((_V * _W,), jnp.int32),
        scratch_types=(
            [pltpu.VMEM((4, 8, _PK_COLS), jnp.float32)] * 2
            + [pltpu.VMEM((_PK_COLS * _W,), jnp.int32)] * 2
            + [pltpu.SemaphoreType.DMA] * 4
        ),
        compiler_params=pltpu.CompilerParams(
            needs_layout_passes=False, use_tc_tiling_on_sc=True),
    )(_sc_pack_body)
    return jnp.reshape(kfn(tbl3, tail3), (_V, _W))


def _sc_scores(cand_hbm, ctx_hbm, u_pos_hbm, ctx_idx_hbm, out_hbm,
               u_idx_a, u_idx_b, idx_a, idx_b, u_rows_a, u_rows_b,
               rows_a, rows_b, scores_a, scores_b,
               s_idx_a, s_idx_b, s_u_a, s_u_b, s_r_a, s_r_b):
    u_idx = (u_idx_a, u_idx_b)
    idx_v = (idx_a, idx_b)
    u_rows = (u_rows_a, u_rows_b)
    rows_v = (rows_a, rows_b)
    scores_v = (scores_a, scores_b)
    s_idx = (s_idx_a, s_idx_b)
    s_u = (s_u_a, s_u_b)
    s_r = (s_r_a, s_r_b)

    wid = lax.axis_index("s") * 2 + lax.axis_index("c")
    lane = lax.iota(jnp.int32, 16)

    def base_of(c):
        return wid * _PER_W + jnp.minimum(c, _CHUNKS - 1) * _E

    def issue_idx(c, b):
        base = base_of(c)
        pltpu.async_copy(u_pos_hbm.at[pl.ds(base, _E)], u_idx[b], s_idx[b])
        pltpu.async_copy(ctx_idx_hbm.at[pl.ds(base, _E)], idx_v[b], s_idx[b])

    def wait_idx(b):
        pltpu.make_async_copy(
            u_pos_hbm.at[pl.ds(0, _E)], u_idx[b], s_idx[b]).wait()
        pltpu.make_async_copy(
            ctx_idx_hbm.at[pl.ds(0, _E)], idx_v[b], s_idx[b]).wait()

    def issue_rows(b):
        pltpu.async_copy(cand_hbm.at[u_idx[b]], u_rows[b], s_u[b])
        for e in range(_E):
            pltpu.async_copy(ctx_hbm.at[idx_v[b].at[e]],
                             rows_v[b].at[pl.ds(e * _R, _RV)], s_r[b])

    def wait_rows(b):
        pltpu.make_async_copy(
            cand_hbm.at[u_idx[b]], u_rows[b], s_u[b]).wait()
        for e in range(_E):
            pltpu.make_async_copy(
                ctx_hbm.at[idx_v[b].at[e]],
                rows_v[b].at[pl.ds(e * _R, _RV)], s_r[b]).wait()

    def compute(c, b):
        for e in range(_E):
            rowids = [jnp.full((16,), e * _R + g * 16, jnp.int32) + lane
                      for g in range(8)]
            e_splat = jnp.full((16,), e, jnp.int32)

            def d_body(d, accs):
                d_splat = jnp.full((16,), d, jnp.int32)
                uw = plsc.load_gather(u_rows[b], [e_splat, d_splat])
                ub = plsc.bitcast(uw, jnp.bfloat16)
                new = []
                for g in range(8):
                    w = plsc.load_gather(rows_v[b], [rowids[g], d_splat])
                    vb = plsc.bitcast(w, jnp.bfloat16)
                    p = vb * ub
                    lo, hi = plsc.unpack(p, format=plsc.PackFormat.INTERLEAVED)
                    new.append(accs[g] + (lo + hi))
                return tuple(new)

            accs = lax.fori_loop(
                0, _W, d_body,
                tuple(jnp.zeros((16,), jnp.float32) for _ in range(8)))
            for g in range(8):
                scores_v[b][e, pl.ds(g * 16, 16)] = accs[g]

        pltpu.sync_copy(scores_v[b], out_hbm.at[pl.ds(base_of(c), _E)])

    issue_idx(0, 0)
    wait_idx(0)
    issue_rows(0)
    issue_idx(1, 1)

    def body(g, _):
        for b in range(2):
            c = 2 * g + b
            wait_idx(1 - b)
            issue_rows(1 - b)
            wait_rows(b)
            issue_idx(c + 2, b)
            compute(c, b)
        return 0

    lax.fori_loop(0, _CHUNKS // 2, body, 0)
    wait_rows(0)
    wait_idx(1)


def _sc_call(cand_embed, ctx_pk, u_pos, ctx_idx):
    kfn = functools.partial(
        pl.kernel,
        mesh=_sc_mesh,
        out_type=jax.ShapeDtypeStruct((_B, _R), jnp.float32),
        scratch_types=(
            [pltpu.VMEM((_E,), jnp.int32)] * 2
            + [pltpu.VMEM((_E, _RV), jnp.int32)] * 2
            + [pltpu.VMEM((_E, _W), jnp.int32)] * 2
            + [pltpu.VMEM((_E * _R, _W), jnp.int32)] * 2
            + [pltpu.VMEM((_E, _R), jnp.float32)] * 2
            + [pltpu.SemaphoreType.DMA] * 6
        ),
        compiler_params=pltpu.CompilerParams(
            needs_layout_passes=False, use_tc_tiling_on_sc=False),
    )(_sc_scores)
    return kfn(cand_embed, ctx_pk, u_pos, ctx_idx)


def _tc_loss_body(scores_ref, vpos_ref, out_ref):
    s = scores_ref[...]
    vp = vpos_ref[...]
    mask = (vp != 0).astype(jnp.float32)

    def logsig(x):
        return jnp.minimum(x, 0.0) - jnp.log1p(jnp.exp(-jnp.abs(x)))

    s_pos = jnp.sum(s[:, :_L] * mask, axis=1)
    s_book = s[:, _L]
    neg = s[:, _L + 1:_L + 1 + 2 * _NNEG]
    loss = -(logsig(s_pos) + logsig(s_book)
             + jnp.sum(logsig(-neg), axis=1))
    out_ref[...] = loss


def _tc_loss(scores, v_pos):
    bs = 2048
    return pl.pallas_call(
        _tc_loss_body,
        grid=(_B // bs,),
        in_specs=[
            pl.BlockSpec((bs, _R), lambda i: (i, 0)),
            pl.BlockSpec((bs, _L), lambda i: (i, 0)),
        ],
        out_specs=pl.BlockSpec((bs,), lambda i: (i,)),
        out_shape=jax.ShapeDtypeStruct((_B,), jnp.float32),
    )(scores, v_pos)


def kernel(u_pos, v_pos, book_pos, v_neg_city, v_neg_country,
           cand_embed, contx_embed):
    ctx_idx = jnp.concatenate(
        [v_pos, book_pos[:, None], v_neg_city, v_neg_country], axis=1)
    cand_pk = _sc_pack(cand_embed)
    ctx_pk = _sc_pack(contx_embed)
    scores = _sc_call(cand_pk, ctx_pk, u_pos, ctx_idx)
    return _tc_loss(scores, v_pos)
